# baseline (speedup 1.0000x reference)
import functools

import jax
import jax.numpy as jnp
from jax import lax
from jax.experimental import pallas as pl
from jax.experimental.pallas import tpu as pltpu
from jax.experimental.pallas import tpu_sc as plsc

_B = 16384
_D = 64
_P = 8192
_PR = 4096
_L1_REG = 0.001
_ALPHA = 0.2

_NC = 2
_NS = 16
_NW = _NC * _NS
_CHUNK = 128

_DIS_A, _DIS_B = 0, _P
_SA_A, _SA_B = 2 * _P, 3 * _P
_REF_A, _REF_B = 4 * _P, 4 * _P + _PR


def _rsqrt_nr(x):
    xi = lax.bitcast_convert_type(x, jnp.int32)
    yi = jnp.int32(0x5F3759DF) - lax.shift_right_logical(xi, 1)
    y = lax.bitcast_convert_type(yi, jnp.float32)
    for _ in range(3):
        y = y * (1.5 - 0.5 * x * y * y)
    return y


def _sc_pair_kernel(comb0, comb1, pidx_hbm, out_hbm,
                    qsets, gsets, outb, sems):
    cid = lax.axis_index("c")
    sid = lax.axis_index("s")
    wid = sid * _NC + cid

    iota16 = lax.iota(jnp.int32, 16)
    zf = jnp.zeros((16,), jnp.float32)

    segs = []
    for b in range(2):
        segs += ([("sa", b, c) for c in range(_P // _NW // _CHUNK)]
                 + [("dis", b, c) for c in range(_P // _NW // _CHUNK)]
                 + [("ref", b, c) for c in range(_PR // _NW // _CHUNK)])

    def seg_pair_offs(kind, c):
        if kind == "dis":
            return (_DIS_A + wid * (_P // _NW) + c * _CHUNK,
                    _DIS_B + wid * (_P // _NW) + c * _CHUNK)
        if kind == "sa":
            return (_SA_A + wid * (_P // _NW) + c * _CHUNK,
                    _SA_B + wid * (_P // _NW) + c * _CHUNK)
        return (_REF_A + wid * (_PR // _NW) + c * _CHUNK,
                _REF_B + wid * (_PR // _NW) + c * _CHUNK)

    def fire(seg, set_id):
        kind, b, c = seg
        st = comb0 if b == 0 else comb1
        qa, qb = qsets[set_id]
        ga, gb = gsets[set_id]
        sem = sems[set_id]
        a_off, b_off = seg_pair_offs(kind, c)
        pltpu.sync_copy(pidx_hbm.at[pl.ds(a_off, _CHUNK)], qa)
        pltpu.sync_copy(pidx_hbm.at[pl.ds(b_off, _CHUNK)], qb)
        return [pltpu.async_copy(st.at[qa], ga, sem),
                pltpu.async_copy(st.at[qb], gb, sem)]

    def sqdist_16(ga, gb, j):
        row = iota16 + j * 16

        @plsc.parallel_loop(0, _D // 16, carry=zf)
        def dc_body(dc, acc):
            a = acc
            base = dc * 16
            for dd in range(16):
                cv = ((iota16 + dd) & 15) + base
                va = plsc.load_gather(ga, [row, cv])
                vb = plsc.load_gather(gb, [row, cv])
                t = va - vb
                a = a + t * t
            return a

        return dc_body

    def sa_quads_16(ga, gb, j):
        row = iota16 + j * 16

        @plsc.parallel_loop(0, _D // 16, carry=(zf, zf, zf, zf))
        def dc_body(dc, carry):
            a_s, a_df, a_qa, a_qb = carry
            base = dc * 16
            for dd in range(16):
                cv = ((iota16 + dd) & 15) + base
                cn = cv + _D
                sa = plsc.load_gather(ga, [row, cv])
                sb = plsc.load_gather(gb, [row, cv])
                na = plsc.load_gather(ga, [row, cn])
                nb = plsc.load_gather(gb, [row, cn])
                dsv = sa - sb
                a_s = a_s + dsv * dsv
                da = na - sa
                db = nb - sb
                dd_ = da - db
                a_df = a_df + dd_ * dd_
                a_qa = a_qa + da * da
                a_qb = a_qb + db * db
            return (a_s, a_df, a_qa, a_qb)

        return dc_body

    def compute(seg, set_id, accs):
        kind = seg[0]
        caus_a, prop_a, rep_a, ref_a = accs
        ga, gb = gsets[set_id]
        if kind == "sa":
            def jbody(j, carry):
                rep_j, prop_j = carry
                a_s, a_df, a_qa, a_qb = sa_quads_16(ga, gb, j)
                sim = jnp.exp(-a_s)
                rep_j = rep_j + sim * a_df
                norm_a = a_qa * _rsqrt_nr(a_qa)
                norm_b = a_qb * _rsqrt_nr(a_qb)
                dn = norm_a - norm_b
                prop_j = prop_j + dn * dn
                return (rep_j, prop_j)

            rep_a, prop_a = lax.fori_loop(0, _CHUNK // 16, jbody,
                                          (rep_a, prop_a))
        elif kind == "dis":
            def jbody(j, caus_j):
                return caus_j + jnp.exp(-sqdist_16(ga, gb, j))

            caus_a = lax.fori_loop(0, _CHUNK // 16, jbody, caus_a)
        else:
            def jbody(j, ref_j):
                return ref_j + sqdist_16(ga, gb, j)

            ref_a = lax.fori_loop(0, _CHUNK // 16, jbody, ref_a)
        return (caus_a, prop_a, rep_a, ref_a)

    accs = (zf, zf, zf, zf)
    hs = fire(segs[0], 0)
    for k, seg in enumerate(segs):
        hs_next = fire(segs[k + 1], (k + 1) % 2) if k + 1 < len(segs) else None
        for h in hs:
            h.wait()
        accs = compute(seg, k % 2, accs)
        hs = hs_next

    caus_acc, prop_acc, rep_acc, ref_acc = accs
    outb[0] = caus_acc
    outb[1] = prop_acc
    outb[2] = rep_acc
    outb[3] = ref_acc
    for k in range(4, 8):
        outb[k] = zf
    pltpu.sync_copy(outb, out_hbm.at[wid])


@functools.partial(
    pl.kernel,
    out_type=jax.ShapeDtypeStruct((_NW, 8, 16), jnp.float32),
    mesh=plsc.VectorSubcoreMesh(core_axis_name="c", subcore_axis_name="s"),
    scratch_types=(
        [pltpu.VMEM((_CHUNK,), jnp.int32)] * 4
        + [pltpu.VMEM((_CHUNK, 128), jnp.float32)] * 4
        + [pltpu.VMEM((8, 16), jnp.float32)]
        + [pltpu.SemaphoreType.DMA] * 2
    ),
    compiler_params=pltpu.CompilerParams(
        use_tc_tiling_on_sc=False,
        needs_layout_passes=False,
    ),
)
def _sc_pairs(comb0, comb1, pidx_hbm, out_hbm,
              qa0, qb0, qa1, qb1,
              ga0, gb0, ga1, gb1,
              outb, sem0, sem1):
    _sc_pair_kernel(
        comb0, comb1, pidx_hbm, out_hbm,
        ((qa0, qb0), (qa1, qb1)),
        ((ga0, gb0), (ga1, gb1)),
        outb, (sem0, sem1))


_CB = 2048


def _comb_kernel(s_ref, n_ref, ps_ref, nps_ref, o0_ref, o1_ref):
    o0_ref[...] = jnp.concatenate([s_ref[...].T, n_ref[...].T], axis=1)
    o1_ref[...] = jnp.concatenate([ps_ref[...].T, nps_ref[...].T], axis=1)


def _comb(states_t, next_states_t, p_states_t, next_p_st_t):
    in_spec = pl.BlockSpec((_D, _CB), lambda i: (0, i))
    out_spec = pl.BlockSpec((_CB, 2 * _D), lambda i: (i, 0))
    o = jax.ShapeDtypeStruct((_B, 2 * _D), jnp.float32)
    return pl.pallas_call(
        _comb_kernel,
        grid=(_B // _CB,),
        in_specs=[in_spec] * 4,
        out_specs=[out_spec] * 2,
        out_shape=[o] * 2,
    )(states_t, next_states_t, p_states_t, next_p_st_t)


def _tc_dense_kernel(s_ref, p_ref, n_ref, ns_ref, nps_ref, w_ref, o_ref):
    s = s_ref[...]
    d1 = ns_ref[...] - s
    t1 = jnp.sum(d1 * d1)
    p = p_ref[...]
    d2 = nps_ref[...] - p
    t2 = jnp.sum(d2 * d2)
    dp = jnp.sum((s - p) ** 2, axis=0)
    dn = jnp.sum((s - n_ref[...]) ** 2, axis=0)
    tri = jnp.sum(jnp.maximum(dp - dn + _ALPHA, 0.0))
    l1 = jnp.where(pl.program_id(0) == 0, jnp.sum(jnp.abs(w_ref[...])), 0.0)
    col = lax.broadcasted_iota(jnp.int32, (1, 1, 128), 2)
    row = jnp.where(col == 0, t1 + t2,
                    jnp.where(col == 1, tri,
                              jnp.where(col == 2, l1, 0.0)))
    o_ref[...] = row


_TC_GRID = 32
_RB = _B // _TC_GRID


def _tc_dense(states_t, p_states_t, n_states_t, next_states_t, next_p_st_t,
              w_t):
    col_spec = pl.BlockSpec((_D, _RB), lambda i: (0, i))
    return pl.pallas_call(
        _tc_dense_kernel,
        grid=(_TC_GRID,),
        in_specs=[col_spec, col_spec, col_spec, col_spec, col_spec,
                  pl.BlockSpec((_D, 256), lambda i: (0, 0))],
        out_specs=pl.BlockSpec((1, 1, 128), lambda i: (i, 0, 0)),
        out_shape=jax.ShapeDtypeStruct((_TC_GRID, 1, 128), jnp.float32),
    )(states_t, p_states_t, n_states_t, next_states_t, next_p_st_t, w_t)


def kernel(states, p_states, n_states, next_states, next_p_st,
           dissimilar_pairs, same_actions_pairs, ref_point_pairs,
           similar_pairs, W):
    del similar_pairs
    dis = dissimilar_pairs.astype(jnp.int32)
    sap = same_actions_pairs.astype(jnp.int32)
    rpp = ref_point_pairs.astype(jnp.int32)
    pidx = jnp.concatenate([dis[:, 0], dis[:, 1], sap[:, 0], sap[:, 1],
                            rpp[:, 0], rpp[:, 1]])

    comb0, comb1 = _comb(states.T, next_states.T, p_states.T, next_p_st.T)

    dense = _tc_dense(states.T, p_states.T, n_states.T, next_states.T,
                      next_p_st.T, W.T)
    partials = _sc_pairs(comb0, comb1, pidx)

    temp_sum = jnp.sum(dense[:, 0, 0])
    tri_sum = jnp.sum(dense[:, 0, 1])
    l1_sum = jnp.sum(dense[:, 0, 2])
    pair_sums = jnp.sum(partials, axis=(0, 2))

    total = (
        (_L1_REG / W.size) * l1_sum
        + temp_sum / _B
        + pair_sums[0] / _P
        + pair_sums[1] / _P
        + pair_sums[2] / _P
        + pair_sums[3] / _PR
        + tri_sum / _B
    )
    return total


# speedup vs baseline: 1.0210x; 1.0210x over previous
import functools

import jax
import jax.numpy as jnp
from jax import lax
from jax.experimental import pallas as pl
from jax.experimental.pallas import tpu as pltpu
from jax.experimental.pallas import tpu_sc as plsc

_B = 16384
_D = 64
_P = 8192
_PR = 4096
_L1_REG = 0.001
_ALPHA = 0.2

_NC = 2
_NS = 16
_NW = _NC * _NS
_CHUNK = 128

_DIS_A, _DIS_B = 0, _P
_SA_A, _SA_B = 2 * _P, 3 * _P
_REF_A, _REF_B = 4 * _P, 4 * _P + _PR


def _rsqrt_nr(x):
    xi = lax.bitcast_convert_type(x, jnp.int32)
    yi = jnp.int32(0x5F3759DF) - lax.shift_right_logical(xi, 1)
    y = lax.bitcast_convert_type(yi, jnp.float32)
    for _ in range(3):
        y = y * (1.5 - 0.5 * x * y * y)
    return y


def _sc_pair_kernel(comb0, comb1, pidx_hbm, out_hbm,
                    qsets, gsets, outb, sems):
    cid = lax.axis_index("c")
    sid = lax.axis_index("s")
    wid = sid * _NC + cid

    iota16 = lax.iota(jnp.int32, 16)
    zf = jnp.zeros((16,), jnp.float32)

    segs = []
    for b in range(2):
        segs += ([("sa", b, c) for c in range(_P // _NW // _CHUNK)]
                 + [("dis", b, c) for c in range(_P // _NW // _CHUNK)]
                 + [("ref", b, c) for c in range(_PR // _NW // _CHUNK)])

    def seg_pair_offs(kind, c):
        if kind == "dis":
            return (_DIS_A + wid * (_P // _NW) + c * _CHUNK,
                    _DIS_B + wid * (_P // _NW) + c * _CHUNK)
        if kind == "sa":
            return (_SA_A + wid * (_P // _NW) + c * _CHUNK,
                    _SA_B + wid * (_P // _NW) + c * _CHUNK)
        return (_REF_A + wid * (_PR // _NW) + c * _CHUNK,
                _REF_B + wid * (_PR // _NW) + c * _CHUNK)

    def fire(seg, set_id):
        kind, b, c = seg
        st = comb0 if b == 0 else comb1
        qa, qb = qsets[set_id]
        ga, gb = gsets[set_id]
        sem = sems[set_id]
        a_off, b_off = seg_pair_offs(kind, c)
        pltpu.sync_copy(pidx_hbm.at[pl.ds(a_off, _CHUNK)], qa)
        pltpu.sync_copy(pidx_hbm.at[pl.ds(b_off, _CHUNK)], qb)
        return [pltpu.async_copy(st.at[qa], ga, sem),
                pltpu.async_copy(st.at[qb], gb, sem)]

    def sqdist_16(ga, gb, j):
        row = iota16 + j * 16

        def dc_body(dc, acc):
            a = acc
            base = dc * 16
            for dd in range(16):
                cv = ((iota16 + dd) & 15) + base
                va = plsc.load_gather(ga, [row, cv])
                vb = plsc.load_gather(gb, [row, cv])
                t = va - vb
                a = a + t * t
            return a

        return lax.fori_loop(0, _D // 16, dc_body, zf)

    def sa_quads_16(ga, gb, j):
        row = iota16 + j * 16

        def dc_body(dc, carry):
            a_s, a_df, a_qa, a_qb = carry
            base = dc * 16
            for dd in range(16):
                cv = ((iota16 + dd) & 15) + base
                cn = cv + _D
                sa = plsc.load_gather(ga, [row, cv])
                sb = plsc.load_gather(gb, [row, cv])
                na = plsc.load_gather(ga, [row, cn])
                nb = plsc.load_gather(gb, [row, cn])
                dsv = sa - sb
                a_s = a_s + dsv * dsv
                da = na - sa
                db = nb - sb
                dd_ = da - db
                a_df = a_df + dd_ * dd_
                a_qa = a_qa + da * da
                a_qb = a_qb + db * db
            return (a_s, a_df, a_qa, a_qb)

        return lax.fori_loop(0, _D // 16, dc_body, (zf, zf, zf, zf))

    def compute(seg, set_id, accs):
        kind = seg[0]
        caus_a, prop_a, rep_a, ref_a = accs
        ga, gb = gsets[set_id]
        if kind == "sa":
            def jbody(j, carry):
                rep_j, prop_j = carry
                a_s, a_df, a_qa, a_qb = sa_quads_16(ga, gb, j)
                sim = jnp.exp(-a_s)
                rep_j = rep_j + sim * a_df
                norm_a = a_qa * _rsqrt_nr(a_qa)
                norm_b = a_qb * _rsqrt_nr(a_qb)
                dn = norm_a - norm_b
                prop_j = prop_j + dn * dn
                return (rep_j, prop_j)

            rep_a, prop_a = lax.fori_loop(0, _CHUNK // 16, jbody,
                                          (rep_a, prop_a))
        elif kind == "dis":
            def jbody(j, caus_j):
                return caus_j + jnp.exp(-sqdist_16(ga, gb, j))

            caus_a = lax.fori_loop(0, _CHUNK // 16, jbody, caus_a)
        else:
            def jbody(j, ref_j):
                return ref_j + sqdist_16(ga, gb, j)

            ref_a = lax.fori_loop(0, _CHUNK // 16, jbody, ref_a)
        return (caus_a, prop_a, rep_a, ref_a)

    nsets = len(gsets)
    accs = (zf, zf, zf, zf)
    pend = [fire(segs[k], k % nsets) for k in range(nsets - 1)]
    for k, seg in enumerate(segs):
        if k + nsets - 1 < len(segs):
            pend.append(fire(segs[k + nsets - 1], (k + nsets - 1) % nsets))
        for h in pend.pop(0):
            h.wait()
        accs = compute(seg, k % nsets, accs)

    caus_acc, prop_acc, rep_acc, ref_acc = accs
    outb[0] = caus_acc
    outb[1] = prop_acc
    outb[2] = rep_acc
    outb[3] = ref_acc
    for k in range(4, 8):
        outb[k] = zf
    pltpu.sync_copy(outb, out_hbm.at[wid])


@functools.partial(
    pl.kernel,
    out_type=jax.ShapeDtypeStruct((_NW, 8, 16), jnp.float32),
    mesh=plsc.VectorSubcoreMesh(core_axis_name="c", subcore_axis_name="s"),
    scratch_types=(
        [pltpu.VMEM((_CHUNK,), jnp.int32)] * 6
        + [pltpu.VMEM((_CHUNK, 128), jnp.float32)] * 6
        + [pltpu.VMEM((8, 16), jnp.float32)]
        + [pltpu.SemaphoreType.DMA] * 3
    ),
    compiler_params=pltpu.CompilerParams(
        use_tc_tiling_on_sc=False,
        needs_layout_passes=False,
    ),
)
def _sc_pairs(comb0, comb1, pidx_hbm, out_hbm,
              qa0, qb0, qa1, qb1, qa2, qb2,
              ga0, gb0, ga1, gb1, ga2, gb2,
              outb, sem0, sem1, sem2):
    _sc_pair_kernel(
        comb0, comb1, pidx_hbm, out_hbm,
        ((qa0, qb0), (qa1, qb1), (qa2, qb2)),
        ((ga0, gb0), (ga1, gb1), (ga2, gb2)),
        outb, (sem0, sem1, sem2))


_CB = 2048


def _comb_kernel(s_ref, n_ref, ps_ref, nps_ref, o0_ref, o1_ref):
    o0_ref[...] = jnp.concatenate([s_ref[...].T, n_ref[...].T], axis=1)
    o1_ref[...] = jnp.concatenate([ps_ref[...].T, nps_ref[...].T], axis=1)


def _comb(states_t, next_states_t, p_states_t, next_p_st_t):
    in_spec = pl.BlockSpec((_D, _CB), lambda i: (0, i))
    out_spec = pl.BlockSpec((_CB, 2 * _D), lambda i: (i, 0))
    o = jax.ShapeDtypeStruct((_B, 2 * _D), jnp.float32)
    return pl.pallas_call(
        _comb_kernel,
        grid=(_B // _CB,),
        in_specs=[in_spec] * 4,
        out_specs=[out_spec] * 2,
        out_shape=[o] * 2,
    )(states_t, next_states_t, p_states_t, next_p_st_t)


def _tc_dense_kernel(s_ref, p_ref, n_ref, ns_ref, nps_ref, w_ref, o_ref):
    s = s_ref[...]
    d1 = ns_ref[...] - s
    t1 = jnp.sum(d1 * d1)
    p = p_ref[...]
    d2 = nps_ref[...] - p
    t2 = jnp.sum(d2 * d2)
    dp = jnp.sum((s - p) ** 2, axis=0)
    dn = jnp.sum((s - n_ref[...]) ** 2, axis=0)
    tri = jnp.sum(jnp.maximum(dp - dn + _ALPHA, 0.0))
    l1 = jnp.where(pl.program_id(0) == 0, jnp.sum(jnp.abs(w_ref[...])), 0.0)
    col = lax.broadcasted_iota(jnp.int32, (1, 1, 128), 2)
    row = jnp.where(col == 0, t1 + t2,
                    jnp.where(col == 1, tri,
                              jnp.where(col == 2, l1, 0.0)))
    o_ref[...] = row


_TC_GRID = 32
_RB = _B // _TC_GRID


def _tc_dense(states_t, p_states_t, n_states_t, next_states_t, next_p_st_t,
              w_t):
    col_spec = pl.BlockSpec((_D, _RB), lambda i: (0, i))
    return pl.pallas_call(
        _tc_dense_kernel,
        grid=(_TC_GRID,),
        in_specs=[col_spec, col_spec, col_spec, col_spec, col_spec,
                  pl.BlockSpec((_D, 256), lambda i: (0, 0))],
        out_specs=pl.BlockSpec((1, 1, 128), lambda i: (i, 0, 0)),
        out_shape=jax.ShapeDtypeStruct((_TC_GRID, 1, 128), jnp.float32),
    )(states_t, p_states_t, n_states_t, next_states_t, next_p_st_t, w_t)


def kernel(states, p_states, n_states, next_states, next_p_st,
           dissimilar_pairs, same_actions_pairs, ref_point_pairs,
           similar_pairs, W):
    del similar_pairs
    dis = dissimilar_pairs.astype(jnp.int32)
    sap = same_actions_pairs.astype(jnp.int32)
    rpp = ref_point_pairs.astype(jnp.int32)
    pidx = jnp.concatenate([dis[:, 0], dis[:, 1], sap[:, 0], sap[:, 1],
                            rpp[:, 0], rpp[:, 1]])

    comb0, comb1 = _comb(states.T, next_states.T, p_states.T, next_p_st.T)

    dense = _tc_dense(states.T, p_states.T, n_states.T, next_states.T,
                      next_p_st.T, W.T)
    partials = _sc_pairs(comb0, comb1, pidx)

    temp_sum = jnp.sum(dense[:, 0, 0])
    tri_sum = jnp.sum(dense[:, 0, 1])
    l1_sum = jnp.sum(dense[:, 0, 2])
    pair_sums = jnp.sum(partials, axis=(0, 2))

    total = (
        (_L1_REG / W.size) * l1_sum
        + temp_sum / _B
        + pair_sums[0] / _P
        + pair_sums[1] / _P
        + pair_sums[2] / _P
        + pair_sums[3] / _PR
        + tri_sum / _B
    )
    return total
